# baseline (speedup 1.0000x reference)
import functools

import jax
import jax.numpy as jnp
from jax import lax
from jax.experimental import pallas as pl
from jax.experimental.pallas import tpu as pltpu
from jax.experimental.pallas import tpu_sc as plsc

N_ROWS = 409600
EMB = 128
CHUNK = 128
K = 2
NW = 32
ROWS_PER_W = N_ROWS // NW
NCHUNK = ROWS_PER_W // CHUNK
G = NCHUNK // K


def _gather_kernel(idx_hbm, table_hbm, out_hbm, idx_v, bufs, gsem0, gsem1,
                   ssem0, ssem1):
  cid = lax.axis_index("c")
  sid = lax.axis_index("s")
  wid = sid * 2 + cid
  row_base = wid * ROWS_PER_W

  pltpu.sync_copy(idx_hbm.at[pl.ds(row_base, ROWS_PER_W)], idx_v)

  gsems = (gsem0, gsem1)
  ssems = (ssem0, ssem1)

  def issue_gathers(g, half):
    for b in range(K):
      j = g * K + b
      pltpu.async_copy(table_hbm.at[idx_v.at[pl.ds(j * CHUNK, CHUNK)]],
                       bufs.at[half * K + b], gsems[half])

  def drain_gathers(half):
    for b in range(K):
      pltpu.make_async_copy(table_hbm.at[idx_v.at[pl.ds(0, CHUNK)]],
                            bufs.at[half * K + b], gsems[half]).wait()

  def issue_stores(g, half):
    for b in range(K):
      j = g * K + b
      dst = out_hbm.at[pl.ds(row_base + j * CHUNK, CHUNK)]
      pltpu.async_copy(bufs.at[half * K + b], dst, ssems[half])

  def drain_stores(half):
    for b in range(K):
      pltpu.make_async_copy(bufs.at[half * K + b],
                            out_hbm.at[pl.ds(row_base, CHUNK)],
                            ssems[half]).wait()

  def process(g, half):
    drain_gathers(half)
    issue_stores(g, half)

  issue_gathers(0, 0)
  issue_gathers(1, 1)

  def body(i, carry):
    g0 = 2 * i
    process(g0, 0)
    drain_stores(0)
    issue_gathers(g0 + 2, 0)
    process(g0 + 1, 1)
    drain_stores(1)
    issue_gathers(g0 + 3, 1)
    return carry

  lax.fori_loop(0, (G - 2) // 2, body, 0)

  process(G - 2, 0)
  drain_stores(0)
  process(G - 1, 1)
  drain_stores(1)


@jax.jit
def _embedding_lookup(x, W):
  idx = (x.reshape(-1)[:N_ROWS] // 2).astype(jnp.int32)
  W2 = W.reshape(500000, 128)
  mesh = plsc.VectorSubcoreMesh(core_axis_name="c", subcore_axis_name="s")
  run = pl.kernel(
      _gather_kernel,
      out_type=jax.ShapeDtypeStruct((N_ROWS, EMB), jnp.float32),
      mesh=mesh,
      scratch_types=[
          pltpu.VMEM((ROWS_PER_W,), jnp.int32),
          pltpu.VMEM((2 * K, CHUNK, EMB), jnp.float32),
          pltpu.SemaphoreType.DMA,
          pltpu.SemaphoreType.DMA,
          pltpu.SemaphoreType.DMA,
          pltpu.SemaphoreType.DMA,
      ],
  )
  out = run(idx, W2)
  return out.reshape(16384, 50, 64)


def kernel(x, W):
  return _embedding_lookup(x, W)


# speedup vs baseline: 1.0019x; 1.0019x over previous
import jax
import jax.numpy as jnp
from jax import lax
from jax.experimental import pallas as pl
from jax.experimental.pallas import tpu as pltpu
from jax.experimental.pallas import tpu_sc as plsc

N_ROWS = 16384 * 50
EMB = 64
CHUNK = 128
K = 4
NW = 32
ROWS_PER_W = N_ROWS // NW
NCHUNK = ROWS_PER_W // CHUNK
G = NCHUNK // K
IDX_HEAD = 2 * K


def _gather_kernel(idx_hbm, table_hbm, out_hbm, idx_v, bufs, gsem0, gsem1,
                   ssem0, ssem1):
  cid = lax.axis_index("c")
  sid = lax.axis_index("s")
  wid = sid * 2 + cid
  idx_base = wid * NCHUNK
  out_base = wid * ROWS_PER_W

  gsems = (gsem0, gsem1)
  ssems = (ssem0, ssem1)

  def issue_gathers(g, half):
    for b in range(K):
      j = g * K + b
      pltpu.async_copy(table_hbm.at[idx_v.at[j]], bufs.at[half * K + b],
                       gsems[half])

  def drain_gathers(half):
    for b in range(K):
      pltpu.make_async_copy(table_hbm.at[idx_v.at[0]], bufs.at[half * K + b],
                            gsems[half]).wait()

  def issue_stores(g, half):
    for b in range(K):
      j = g * K + b
      dst = out_hbm.at[pl.ds(out_base + j * CHUNK, CHUNK)]
      pltpu.async_copy(bufs.at[half * K + b], dst, ssems[half])

  def drain_stores(half):
    for b in range(K):
      pltpu.make_async_copy(bufs.at[half * K + b],
                            out_hbm.at[pl.ds(out_base, CHUNK)],
                            ssems[half]).wait()

  def process(g, half):
    drain_gathers(half)
    issue_stores(g, half)

  pltpu.sync_copy(idx_hbm.at[pl.ds(idx_base, IDX_HEAD)],
                  idx_v.at[pl.ds(0, IDX_HEAD)])
  issue_gathers(0, 0)
  issue_gathers(1, 1)
  pltpu.sync_copy(idx_hbm.at[pl.ds(idx_base + IDX_HEAD, NCHUNK - IDX_HEAD)],
                  idx_v.at[pl.ds(IDX_HEAD, NCHUNK - IDX_HEAD)])

  def body(i, carry):
    g0 = 2 * i
    process(g0, 0)
    drain_stores(0)
    issue_gathers(g0 + 2, 0)
    process(g0 + 1, 1)
    drain_stores(1)
    issue_gathers(g0 + 3, 1)
    return carry

  lax.fori_loop(0, (G - 2) // 2, body, 0)

  process(G - 2, 0)
  drain_stores(0)
  process(G - 1, 1)
  drain_stores(1)


@jax.jit
def _embedding_lookup(x, W):
  idx = x.reshape(N_ROWS // CHUNK, CHUNK).astype(jnp.int32)
  mesh = plsc.VectorSubcoreMesh(core_axis_name="c", subcore_axis_name="s")
  run = pl.kernel(
      _gather_kernel,
      out_type=jax.ShapeDtypeStruct((N_ROWS, EMB), jnp.float32),
      mesh=mesh,
      scratch_types=[
          pltpu.VMEM((NCHUNK, CHUNK), jnp.int32),
          pltpu.VMEM((2 * K, CHUNK, EMB), jnp.float32),
          pltpu.SemaphoreType.DMA,
          pltpu.SemaphoreType.DMA,
          pltpu.SemaphoreType.DMA,
          pltpu.SemaphoreType.DMA,
      ],
      compiler_params=pltpu.CompilerParams(use_tc_tiling_on_sc=False),
  )
  out = run(idx, W)
  return out.reshape(x.shape[0], x.shape[1], EMB)


def kernel(x, W):
  return _embedding_lookup(x, W)


# speedup vs baseline: 1.0023x; 1.0004x over previous
import jax
import jax.numpy as jnp
from jax import lax
from jax.experimental import pallas as pl
from jax.experimental.pallas import tpu as pltpu
from jax.experimental.pallas import tpu_sc as plsc

N_ROWS = 16384 * 50
EMB = 64
CHUNK = 128
K = 4
NW = 32
ROWS_PER_W = N_ROWS // NW
NCHUNK = ROWS_PER_W // CHUNK
G = NCHUNK // K
IDX_HEAD = 2 * K


def _gather_kernel(idx_hbm, table_hbm, out_hbm, idx_v, bufs, *sems):
  gsems = sems[:2 * K]
  ssems = sems[2 * K:]
  cid = lax.axis_index("c")
  sid = lax.axis_index("s")
  wid = sid * 2 + cid
  idx_base = wid * NCHUNK
  out_base = wid * ROWS_PER_W

  def issue_gathers(g, half):
    for b in range(K):
      j = g * K + b
      pltpu.async_copy(table_hbm.at[idx_v.at[j]], bufs.at[half * K + b],
                       gsems[half * K + b])

  def issue_stores(g, half):
    for b in range(K):
      j = g * K + b
      pltpu.make_async_copy(table_hbm.at[idx_v.at[0]], bufs.at[half * K + b],
                            gsems[half * K + b]).wait()
      dst = out_hbm.at[pl.ds(out_base + j * CHUNK, CHUNK)]
      pltpu.async_copy(bufs.at[half * K + b], dst, ssems[half])

  def drain_stores(half):
    for b in range(K):
      pltpu.make_async_copy(bufs.at[half * K + b],
                            out_hbm.at[pl.ds(out_base, CHUNK)],
                            ssems[half]).wait()

  pltpu.sync_copy(idx_hbm.at[pl.ds(idx_base, IDX_HEAD)],
                  idx_v.at[pl.ds(0, IDX_HEAD)])
  issue_gathers(0, 0)
  issue_gathers(1, 1)
  pltpu.sync_copy(idx_hbm.at[pl.ds(idx_base + IDX_HEAD, NCHUNK - IDX_HEAD)],
                  idx_v.at[pl.ds(IDX_HEAD, NCHUNK - IDX_HEAD)])

  def body(i, carry):
    g0 = 2 * i
    issue_stores(g0, 0)
    drain_stores(0)
    issue_gathers(g0 + 2, 0)
    issue_stores(g0 + 1, 1)
    drain_stores(1)
    issue_gathers(g0 + 3, 1)
    return carry

  lax.fori_loop(0, (G - 2) // 2, body, 0)

  issue_stores(G - 2, 0)
  drain_stores(0)
  issue_stores(G - 1, 1)
  drain_stores(1)


@jax.jit
def _embedding_lookup(x, W):
  idx = x.reshape(N_ROWS // CHUNK, CHUNK).astype(jnp.int32)
  mesh = plsc.VectorSubcoreMesh(core_axis_name="c", subcore_axis_name="s")
  run = pl.kernel(
      _gather_kernel,
      out_type=jax.ShapeDtypeStruct((N_ROWS, EMB), jnp.float32),
      mesh=mesh,
      scratch_types=(
          [pltpu.VMEM((NCHUNK, CHUNK), jnp.int32),
           pltpu.VMEM((2 * K, CHUNK, EMB), jnp.float32)]
          + [pltpu.SemaphoreType.DMA] * (2 * K)
          + [pltpu.SemaphoreType.DMA] * 2
      ),
      compiler_params=pltpu.CompilerParams(use_tc_tiling_on_sc=False),
  )
  out = run(idx, W)
  return out.reshape(x.shape[0], x.shape[1], EMB)


def kernel(x, W):
  return _embedding_lookup(x, W)
